# Initial kernel scaffold; baseline (speedup 1.0000x reference)
import functools

import jax
import jax.numpy as jnp
from jax import lax
from jax.experimental import pallas as pl
from jax.experimental.pallas import tpu as pltpu
from jax.experimental.pallas import tpu_sc as plsc

N = 10000
E = 320000
D = 128
R = 16

NC = 2
NS = 16
CHUNK = 128
NCHUNKS_PER_SC = (E // NC) // CHUNK
ITERS = (NCHUNKS_PER_SC + NS - 1) // NS
ROWS_PER_TILE = N // NS


def _sc_body(h_hbm, src_hbm, dst_hbm, attr_hbm,
             out_h, out_a, out_d,
             srcv, dstv, rows, attrv, zbuf, onesbuf,
             hacc, aacc, dacc):
    c = lax.axis_index("c")
    s = lax.axis_index("s")

    zero16 = jnp.zeros((16,), jnp.float32)
    one16 = jnp.ones((16,), jnp.float32)

    def fill_row(i, _):
        for j in range(D // 16):
            zbuf[i, pl.ds(j * 16, 16)] = zero16
        onesbuf[i, :] = one16
        return 0

    lax.fori_loop(0, CHUNK, fill_row, 0)

    base = s * ROWS_PER_TILE
    off = 0
    for n in (128, 128, 128, 128, ROWS_PER_TILE - 4 * 128):
        pltpu.sync_copy(zbuf.at[pl.ds(0, n)], hacc.at[pl.ds(base + off, n)])
        pltpu.sync_copy(zbuf.at[pl.ds(0, n), pl.ds(0, R)],
                        aacc.at[pl.ds(base + off, n)])
        pltpu.sync_copy(zbuf.at[pl.ds(0, n), pl.ds(0, 16)],
                        dacc.at[pl.ds(base + off, n)])
        off += n
    plsc.subcore_barrier()

    def chunk_body(k, _):
        cidx = s + k * NS

        @pl.when(cidx < NCHUNKS_PER_SC)
        def _():
            e0 = c * (E // NC) + cidx * CHUNK
            pltpu.sync_copy(src_hbm.at[pl.ds(e0, CHUNK)], srcv)
            pltpu.sync_copy(dst_hbm.at[pl.ds(e0, CHUNK)], dstv)
            pltpu.sync_copy(attr_hbm.at[pl.ds(e0, CHUNK)], attrv)
            pltpu.sync_copy(h_hbm.at[srcv], rows)
            pltpu.sync_copy(rows, hacc.at[dstv], add=True)
            pltpu.sync_copy(attrv, aacc.at[dstv], add=True)
            pltpu.sync_copy(onesbuf, dacc.at[dstv], add=True)

        return 0

    lax.fori_loop(0, ITERS, chunk_body, 0)
    plsc.subcore_barrier()

    pltpu.sync_copy(hacc.at[pl.ds(base, ROWS_PER_TILE)],
                    out_h.at[c, pl.ds(base, ROWS_PER_TILE)])
    pltpu.sync_copy(aacc.at[pl.ds(base, ROWS_PER_TILE)],
                    out_a.at[c, pl.ds(base, ROWS_PER_TILE)])
    pltpu.sync_copy(dacc.at[pl.ds(base, ROWS_PER_TILE)],
                    out_d.at[c, pl.ds(base, ROWS_PER_TILE)])


_sc_segment_sums = functools.partial(
    pl.kernel,
    out_type=(
        jax.ShapeDtypeStruct((NC, N, D), jnp.float32),
        jax.ShapeDtypeStruct((NC, N, R), jnp.float32),
        jax.ShapeDtypeStruct((NC, N, 16), jnp.float32),
    ),
    mesh=plsc.VectorSubcoreMesh(core_axis_name="c", subcore_axis_name="s"),
    scratch_types=(
        pltpu.VMEM((CHUNK,), jnp.int32),
        pltpu.VMEM((CHUNK,), jnp.int32),
        pltpu.VMEM((CHUNK, D), jnp.float32),
        pltpu.VMEM((CHUNK, R), jnp.float32),
        pltpu.VMEM((CHUNK, D), jnp.float32),
        pltpu.VMEM((CHUNK, 16), jnp.float32),
        pltpu.VMEM_SHARED((N, D), jnp.float32),
        pltpu.VMEM_SHARED((N, R), jnp.float32),
        pltpu.VMEM_SHARED((N, 16), jnp.float32),
    ),
)(_sc_body)


def _tc_body(h, q, hs2, as2, dg2,
             W_query, b_query, W_mem, b_mem, Wm1, Wm2, b_msg,
             Wa1, Wa2, Wa3, b_all, out):
    hp = lax.dot(hs2[0] + hs2[1], Wm1[...], precision=lax.Precision.HIGHEST)
    ap = lax.dot(as2[0] + as2[1], Wm2[...], precision=lax.Precision.HIGHEST)
    dg = dg2[0, :, 0:1] + dg2[1, :, 0:1]
    agg = hp + ap + dg * b_msg[...]
    t = lax.dot(q[...], W_query[...], precision=lax.Precision.HIGHEST) \
        + b_query[...]
    u = lax.dot(h[...], W_mem[...], precision=lax.Precision.HIGHEST) \
        + b_mem[...]
    pre = (lax.dot(t, Wa1[...], precision=lax.Precision.HIGHEST)
           + lax.dot(u, Wa2[...], precision=lax.Precision.HIGHEST)
           + lax.dot(agg, Wa3[...], precision=lax.Precision.HIGHEST)
           + b_all[...])
    new_h = jnp.maximum(pre, 0.0)
    out[...] = jnp.where(dg > 0.0, new_h, h[...])


def kernel(h, q, edge_index, edge_attr, W_msg, b_msg, W_mem, b_mem,
           W_query, b_query, W_all, b_all):
    edge_index = edge_index.astype(jnp.int32)
    src = edge_index[0]
    dst = edge_index[1]

    hs2, as2, dg2 = _sc_segment_sums(h, src, dst, edge_attr)

    Wm1 = W_msg[:D]
    Wm2 = W_msg[D:]
    Wa1 = W_all[:D]
    Wa2 = W_all[D:2 * D]
    Wa3 = W_all[2 * D:]

    BN = 1000
    grid = N // BN
    row_block = lambda r, c: pl.BlockSpec((r, c), lambda i: (i, 0))
    full = lambda *shape: pl.BlockSpec(shape, lambda i: (0,) * len(shape))

    out = pl.pallas_call(
        _tc_body,
        grid=(grid,),
        in_specs=[
            row_block(BN, D),
            row_block(BN, D),
            pl.BlockSpec((NC, BN, D), lambda i: (0, i, 0)),
            pl.BlockSpec((NC, BN, R), lambda i: (0, i, 0)),
            pl.BlockSpec((NC, BN, 16), lambda i: (0, i, 0)),
            full(D, D),
            full(1, D),
            full(D, D),
            full(1, D),
            full(D, D),
            full(R, D),
            full(1, D),
            full(D, D),
            full(D, D),
            full(D, D),
            full(1, D),
        ],
        out_specs=row_block(BN, D),
        out_shape=jax.ShapeDtypeStruct((N, D), jnp.float32),
    )(h, q, hs2, as2, dg2,
      W_query, b_query.reshape(1, D), W_mem, b_mem.reshape(1, D),
      Wm1, Wm2, b_msg.reshape(1, D),
      Wa1, Wa2, Wa3, b_all.reshape(1, D))
    return out


# speedup vs baseline: 2.7433x; 2.7433x over previous
import functools

import jax
import jax.numpy as jnp
from jax import lax
from jax.experimental import pallas as pl
from jax.experimental.pallas import tpu as pltpu
from jax.experimental.pallas import tpu_sc as plsc

N = 10000
E = 320000
D = 128
HD = D // 2
R = 16

NC = 2
NS = 16
CHUNK = 128
NCHUNKS = E // CHUNK
ITERS = (NCHUNKS + NS - 1) // NS
N_PAD = 10240
ROWS_PER_TILE = N_PAD // NS


def _sc_body(ha_hbm, hb_hbm, src_hbm, dst_hbm, attr_hbm,
             out_h, out_a, out_d,
             srcv, dstv, rows, attrv, zbuf, zbuf16, onesbuf,
             hacc, sacc):
    c = lax.axis_index("c")
    s = lax.axis_index("s")

    zero16 = jnp.zeros((16,), jnp.float32)
    one16 = jnp.ones((16,), jnp.float32)

    def fill_row(i, _):
        for j in range(HD // 16):
            zbuf[i, pl.ds(j * 16, 16)] = zero16
        zbuf16[i, :] = zero16
        onesbuf[i, :] = one16
        return 0

    lax.fori_loop(0, CHUNK, fill_row, 0)

    base = s * ROWS_PER_TILE
    for k in range(ROWS_PER_TILE // CHUNK):
        off = k * CHUNK
        pltpu.sync_copy(zbuf, hacc.at[pl.ds(base + off, CHUNK)])
        pltpu.sync_copy(zbuf16, sacc.at[pl.ds(base + off, CHUNK)])
    plsc.subcore_barrier()

    def run_edges(h_half, use_attr):
        def chunk_body(k, _):
            cidx = s + k * NS

            @pl.when(cidx < NCHUNKS)
            def _():
                e0 = cidx * CHUNK
                pltpu.sync_copy(src_hbm.at[pl.ds(e0, CHUNK)], srcv)
                pltpu.sync_copy(dst_hbm.at[pl.ds(e0, CHUNK)], dstv)
                pltpu.sync_copy(h_half.at[srcv], rows)
                pltpu.sync_copy(rows, hacc.at[dstv], add=True)
                if use_attr:
                    pltpu.sync_copy(attr_hbm.at[pl.ds(e0, CHUNK)], attrv)
                    pltpu.sync_copy(attrv, sacc.at[dstv], add=True)
                else:
                    pltpu.sync_copy(onesbuf, sacc.at[dstv], add=True)

            return 0

        lax.fori_loop(0, ITERS, chunk_body, 0)

    @pl.when(c == 0)
    def _():
        run_edges(ha_hbm, True)

    @pl.when(c == 1)
    def _():
        run_edges(hb_hbm, False)

    plsc.subcore_barrier()

    hs_src = hacc.at[pl.ds(base, ROWS_PER_TILE)]
    ss_src = sacc.at[pl.ds(base, ROWS_PER_TILE)]

    @pl.when(c == 0)
    def _():
        pltpu.sync_copy(hs_src, out_h.at[0, pl.ds(base, ROWS_PER_TILE)])
        pltpu.sync_copy(ss_src, out_a.at[pl.ds(base, ROWS_PER_TILE)])

    @pl.when(c == 1)
    def _():
        pltpu.sync_copy(hs_src, out_h.at[1, pl.ds(base, ROWS_PER_TILE)])
        pltpu.sync_copy(ss_src, out_d.at[pl.ds(base, ROWS_PER_TILE)])


@functools.cache
def _make_sc_segment_sums():
  return functools.partial(
    pl.kernel,
    out_type=(
        jax.ShapeDtypeStruct((NC, N_PAD, HD), jnp.float32),
        jax.ShapeDtypeStruct((N_PAD, R), jnp.float32),
        jax.ShapeDtypeStruct((N_PAD, 16), jnp.float32),
    ),
    mesh=plsc.VectorSubcoreMesh(core_axis_name="c", subcore_axis_name="s"),
    compiler_params=pltpu.CompilerParams(use_tc_tiling_on_sc=False),
    scratch_types=(
        pltpu.VMEM((CHUNK,), jnp.int32),
        pltpu.VMEM((CHUNK,), jnp.int32),
        pltpu.VMEM((CHUNK, HD), jnp.float32),
        pltpu.VMEM((CHUNK, R), jnp.float32),
        pltpu.VMEM((CHUNK, HD), jnp.float32),
        pltpu.VMEM((CHUNK, 16), jnp.float32),
        pltpu.VMEM((CHUNK, 16), jnp.float32),
        pltpu.VMEM_SHARED((N_PAD, HD), jnp.float32),
        pltpu.VMEM_SHARED((N_PAD, 16), jnp.float32),
    ),
  )(_sc_body)


def _tc_body(h, q, hs2, as_, dg_,
             W_query, b_query, W_mem, b_mem, Wm1, Wm2, b_msg,
             Wa1, Wa2, Wa3, b_all, out):
    hs = jnp.concatenate([hs2[0], hs2[1]], axis=-1)
    hp = lax.dot(hs, Wm1[...], precision=lax.Precision.HIGHEST)
    ap = lax.dot(as_[...], Wm2[...], precision=lax.Precision.HIGHEST)
    dg = dg_[:, 0:1]
    agg = hp + ap + dg * b_msg[...]
    t = lax.dot(q[...], W_query[...], precision=lax.Precision.HIGHEST) \
        + b_query[...]
    u = lax.dot(h[...], W_mem[...], precision=lax.Precision.HIGHEST) \
        + b_mem[...]
    pre = (lax.dot(t, Wa1[...], precision=lax.Precision.HIGHEST)
           + lax.dot(u, Wa2[...], precision=lax.Precision.HIGHEST)
           + lax.dot(agg, Wa3[...], precision=lax.Precision.HIGHEST)
           + b_all[...])
    new_h = jnp.maximum(pre, 0.0)
    out[...] = jnp.where(dg > 0.0, new_h, h[...])


def kernel(h, q, edge_index, edge_attr, W_msg, b_msg, W_mem, b_mem,
           W_query, b_query, W_all, b_all):
    edge_index = edge_index.astype(jnp.int32)
    src = edge_index[0]
    dst = edge_index[1]
    ha = h[:, :HD]
    hb = h[:, HD:]

    hs2, as_, dg_ = _make_sc_segment_sums()(ha, hb, src, dst, edge_attr)

    Wm1 = W_msg[:D]
    Wm2 = W_msg[D:]
    Wa1 = W_all[:D]
    Wa2 = W_all[D:2 * D]
    Wa3 = W_all[2 * D:]

    BN = 1000
    grid = N // BN
    row_block = lambda r, c: pl.BlockSpec((r, c), lambda i: (i, 0))
    full = lambda *shape: pl.BlockSpec(shape, lambda i: (0,) * len(shape))

    out = pl.pallas_call(
        _tc_body,
        grid=(grid,),
        in_specs=[
            row_block(BN, D),
            row_block(BN, D),
            pl.BlockSpec((NC, BN, HD), lambda i: (0, i, 0)),
            row_block(BN, R),
            row_block(BN, 16),
            full(D, D),
            full(1, D),
            full(D, D),
            full(1, D),
            full(D, D),
            full(R, D),
            full(1, D),
            full(D, D),
            full(D, D),
            full(D, D),
            full(1, D),
        ],
        out_specs=row_block(BN, D),
        out_shape=jax.ShapeDtypeStruct((N, D), jnp.float32),
    )(h, q, hs2, as_, dg_,
      W_query, b_query.reshape(1, D), W_mem, b_mem.reshape(1, D),
      Wm1, Wm2, b_msg.reshape(1, D),
      Wa1, Wa2, Wa3, b_all.reshape(1, D))
    return out


# speedup vs baseline: 4.8288x; 1.7602x over previous
import functools

import jax
import jax.numpy as jnp
from jax import lax
from jax.experimental import pallas as pl
from jax.experimental.pallas import tpu as pltpu
from jax.experimental.pallas import tpu_sc as plsc

N = 10000
E = 320000
D = 128
HD = D // 2
R = 16

NC = 2
NS = 16
CHUNK = 128
NCHUNKS = E // CHUNK
CHUNKS_MAIN = NCHUNKS // NS
NLEFT = NCHUNKS - NS * CHUNKS_MAIN
N_PAD = 10240
ROWS_PER_TILE = N_PAD // NS


def _sc_body(ha_hbm, hb_hbm, src_hbm, dst_hbm, attr_hbm,
             out_h, out_a, out_d,
             srcall, dstall, rows0, rows1, attrv0, attrv1,
             sema0, sema1, semg0, semg1, sems0, sems1,
             hacc, sacc):
    c = lax.axis_index("c")
    s = lax.axis_index("s")
    rows = (rows0, rows1)
    attrv = (attrv0, attrv1)
    sema = (sema0, sema1)
    semg = (semg0, semg1)
    sems = (sems0, sems1)

    zero16 = jnp.zeros((16,), jnp.float32)
    one16 = jnp.ones((16,), jnp.float32)

    def fill_zero(i, _):
        for j in range(HD // 16):
            rows0[i, pl.ds(j * 16, 16)] = zero16
        attrv0[i, :] = zero16
        return 0

    lax.fori_loop(0, CHUNK, fill_zero, 0)

    base = s * ROWS_PER_TILE
    for k in range(ROWS_PER_TILE // CHUNK):
        off = k * CHUNK
        pltpu.sync_copy(rows0, hacc.at[pl.ds(base + off, CHUNK)])
        pltpu.sync_copy(attrv0, sacc.at[pl.ds(base + off, CHUNK)])

    @pl.when(c == 1)
    def _():
        def fill_ones(i, _):
            attrv0[i, :] = one16
            return 0

        lax.fori_loop(0, CHUNK, fill_ones, 0)

    chunk0 = s * CHUNKS_MAIN
    pltpu.sync_copy(src_hbm.at[pl.ds(chunk0, CHUNKS_MAIN)], srcall)
    pltpu.sync_copy(dst_hbm.at[pl.ds(chunk0, CHUNKS_MAIN)], dstall)
    plsc.subcore_barrier()

    def run_pipeline(h_half, use_attr):
        edge_base = s * (CHUNKS_MAIN * CHUNK)

        def fire_step(g, b):
            if use_attr:
                da = pltpu.async_copy(
                    attr_hbm.at[pl.ds(edge_base + g * CHUNK, CHUNK)],
                    attrv[b], sema[b])
            gd = pltpu.async_copy(h_half.at[srcall.at[g]], rows[b], semg[b])
            gd.wait()
            descs = [
                pltpu.async_copy(rows[b], hacc.at[dstall.at[g]], sems[b],
                                 add=True),
            ]
            if use_attr:
                da.wait()
                descs.append(pltpu.async_copy(
                    attrv[b], sacc.at[dstall.at[g]], sems[b], add=True))
            else:
                descs.append(pltpu.async_copy(
                    attrv0, sacc.at[dstall.at[g]], sems[b], add=True))
            return descs

        drain = [None, None]
        drain[0] = fire_step(0, 0)
        drain[1] = fire_step(1, 1)

        @pl.loop(2, CHUNKS_MAIN, step=2)
        def _(g):
            for b in range(2):
                for d in drain[b]:
                    d.wait()
                fire_step(g + b, b)

        for d in drain[0]:
            d.wait()
        for d in drain[1]:
            d.wait()

        @pl.when(s < NLEFT)
        def _():
            xchunk = NS * CHUNKS_MAIN + s
            pltpu.sync_copy(src_hbm.at[pl.ds(xchunk, 1)],
                            srcall.at[pl.ds(0, 1)])
            pltpu.sync_copy(dst_hbm.at[pl.ds(xchunk, 1)],
                            dstall.at[pl.ds(0, 1)])
            pltpu.sync_copy(h_half.at[srcall.at[0]], rows0)
            pltpu.sync_copy(rows0, hacc.at[dstall.at[0]], add=True)
            if use_attr:
                pltpu.sync_copy(attr_hbm.at[pl.ds(xchunk * CHUNK, CHUNK)],
                                attrv1)
                pltpu.sync_copy(attrv1, sacc.at[dstall.at[0]], add=True)
            else:
                pltpu.sync_copy(attrv0, sacc.at[dstall.at[0]], add=True)

    @pl.when(c == 0)
    def _():
        run_pipeline(ha_hbm, True)

    @pl.when(c == 1)
    def _():
        run_pipeline(hb_hbm, False)

    plsc.subcore_barrier()

    hs_src = hacc.at[pl.ds(base, ROWS_PER_TILE)]
    ss_src = sacc.at[pl.ds(base, ROWS_PER_TILE)]

    @pl.when(c == 0)
    def _():
        pltpu.sync_copy(hs_src, out_h.at[0, pl.ds(base, ROWS_PER_TILE)])
        pltpu.sync_copy(ss_src, out_a.at[pl.ds(base, ROWS_PER_TILE)])

    @pl.when(c == 1)
    def _():
        pltpu.sync_copy(hs_src, out_h.at[1, pl.ds(base, ROWS_PER_TILE)])
        pltpu.sync_copy(ss_src, out_d.at[pl.ds(base, ROWS_PER_TILE)])


@functools.cache
def _make_sc_segment_sums():
  return functools.partial(
    pl.kernel,
    out_type=(
        jax.ShapeDtypeStruct((NC, N_PAD, HD), jnp.float32),
        jax.ShapeDtypeStruct((N_PAD, R), jnp.float32),
        jax.ShapeDtypeStruct((N_PAD, 16), jnp.float32),
    ),
    mesh=plsc.VectorSubcoreMesh(core_axis_name="c", subcore_axis_name="s"),
    compiler_params=pltpu.CompilerParams(use_tc_tiling_on_sc=False),
    scratch_types=(
        pltpu.VMEM((CHUNKS_MAIN, CHUNK), jnp.int32),
        pltpu.VMEM((CHUNKS_MAIN, CHUNK), jnp.int32),
        pltpu.VMEM((CHUNK, HD), jnp.float32),
        pltpu.VMEM((CHUNK, HD), jnp.float32),
        pltpu.VMEM((CHUNK, R), jnp.float32),
        pltpu.VMEM((CHUNK, R), jnp.float32),
        pltpu.SemaphoreType.DMA,
        pltpu.SemaphoreType.DMA,
        pltpu.SemaphoreType.DMA,
        pltpu.SemaphoreType.DMA,
        pltpu.SemaphoreType.DMA,
        pltpu.SemaphoreType.DMA,
        pltpu.VMEM_SHARED((N_PAD, HD), jnp.float32),
        pltpu.VMEM_SHARED((N_PAD, 16), jnp.float32),
    ),
  )(_sc_body)


def _tc_body(h, q, hs2, as_, dg_,
             W_query, b_query, W_mem, b_mem, Wm1, Wm2, b_msg,
             Wa1, Wa2, Wa3, b_all, out):
    hs = jnp.concatenate([hs2[0], hs2[1]], axis=-1)
    hp = lax.dot(hs, Wm1[...], precision=lax.Precision.HIGHEST)
    ap = lax.dot(as_[...], Wm2[...], precision=lax.Precision.HIGHEST)
    dg = dg_[:, 0:1]
    agg = hp + ap + dg * b_msg[...]
    t = lax.dot(q[...], W_query[...], precision=lax.Precision.HIGHEST) \
        + b_query[...]
    u = lax.dot(h[...], W_mem[...], precision=lax.Precision.HIGHEST) \
        + b_mem[...]
    pre = (lax.dot(t, Wa1[...], precision=lax.Precision.HIGHEST)
           + lax.dot(u, Wa2[...], precision=lax.Precision.HIGHEST)
           + lax.dot(agg, Wa3[...], precision=lax.Precision.HIGHEST)
           + b_all[...])
    new_h = jnp.maximum(pre, 0.0)
    out[...] = jnp.where(dg > 0.0, new_h, h[...])


def kernel(h, q, edge_index, edge_attr, W_msg, b_msg, W_mem, b_mem,
           W_query, b_query, W_all, b_all):
    edge_index = edge_index.astype(jnp.int32)
    src2d = edge_index[0].reshape(E // CHUNK, CHUNK)
    dst2d = edge_index[1].reshape(E // CHUNK, CHUNK)
    ha = h[:, :HD]
    hb = h[:, HD:]

    hs2, as_, dg_ = _make_sc_segment_sums()(ha, hb, src2d, dst2d, edge_attr)

    Wm1 = W_msg[:D]
    Wm2 = W_msg[D:]
    Wa1 = W_all[:D]
    Wa2 = W_all[D:2 * D]
    Wa3 = W_all[2 * D:]

    BN = 1000
    grid = N // BN
    row_block = lambda r, c: pl.BlockSpec((r, c), lambda i: (i, 0))
    full = lambda *shape: pl.BlockSpec(shape, lambda i: (0,) * len(shape))

    out = pl.pallas_call(
        _tc_body,
        grid=(grid,),
        in_specs=[
            row_block(BN, D),
            row_block(BN, D),
            pl.BlockSpec((NC, BN, HD), lambda i: (0, i, 0)),
            row_block(BN, R),
            row_block(BN, 16),
            full(D, D),
            full(1, D),
            full(D, D),
            full(1, D),
            full(D, D),
            full(R, D),
            full(1, D),
            full(D, D),
            full(D, D),
            full(D, D),
            full(1, D),
        ],
        out_specs=row_block(BN, D),
        out_shape=jax.ShapeDtypeStruct((N, D), jnp.float32),
    )(h, q, hs2, as_, dg_,
      W_query, b_query.reshape(1, D), W_mem, b_mem.reshape(1, D),
      Wm1, Wm2, b_msg.reshape(1, D),
      Wa1, Wa2, Wa3, b_all.reshape(1, D))
    return out
